# Initial kernel scaffold; baseline (speedup 1.0000x reference)
import jax
import jax.numpy as jnp
from jax.experimental import pallas as pl


def kernel(x, edge_index, edge_feat, W1, b1, W2, b2):
    raise NotImplementedError("write your pallas kernel here")


# speedup vs baseline: 15.4034x; 15.4034x over previous
import functools

import jax
import jax.numpy as jnp
from jax import lax
from jax.experimental import pallas as pl
from jax.experimental.pallas import tpu as pltpu
from jax.experimental.pallas import tpu_sc as plsc

N = 10000
E = 320000
D = 128

NTILES = 16
NSL = 640
NPAD = NTILES * NSL
CH = 128
E2 = E + N
EPT = 20736
EPAD = NTILES * EPT
NCH = EPT // CH


def _sc_body(xp, rowp, colp, ewp, out, agg_sh, deg_sh, dinv_sh,
             row_b, col_b, ew_b, aw_b, coeff_b, dinv_loc, xb, sem):
    cid = lax.axis_index("c")
    sid = lax.axis_index("s")
    zero16 = jnp.zeros((16,), jnp.float32)

    def _zero_xb(i, c):
        for j in range(D // 16):
            xb[i, pl.ds(j * 16, 16)] = zero16
        return c
    lax.fori_loop(0, CH, _zero_xb, 0)
    for j in range(CH // 16):
        aw_b[pl.ds(j * 16, 16)] = zero16
    for k in range(NSL // CH):
        pltpu.sync_copy(xb, agg_sh.at[pl.ds(sid * NSL + k * CH, CH)])
        pltpu.sync_copy(aw_b, deg_sh.at[pl.ds(sid * NSL + k * CH, CH)])
    plsc.subcore_barrier()

    def _deg_chunk(ch, c):
        base = sid * EPT + ch * CH
        pltpu.sync_copy(colp.at[pl.ds(base, CH)], col_b)
        pltpu.sync_copy(ewp.at[cid, pl.ds(base, CH)], ew_b)
        for j in range(CH // 16):
            s = pl.ds(j * 16, 16)
            aw_b[s] = jnp.abs(ew_b[s])
        pltpu.sync_copy(aw_b, deg_sh.at[col_b], add=True)
        return c
    lax.fori_loop(0, NCH, _deg_chunk, 0)
    plsc.subcore_barrier()

    sl = pl.ds(sid * NSL, NSL)
    pltpu.sync_copy(deg_sh.at[sl], dinv_loc.at[pl.ds(0, NSL)])

    def _rsqrt_vreg(i, c):
        s = pl.ds(i * 16, 16)
        d = dinv_loc[s]
        ii = lax.bitcast_convert_type(d, jnp.int32)
        ii = jnp.int32(0x5F3759DF) - (ii >> 1)
        y = lax.bitcast_convert_type(ii, jnp.float32)
        for _ in range(3):
            y = y * (1.5 - 0.5 * d * y * y)
        dinv_loc[s] = y
        return c
    lax.fori_loop(0, NSL // 16, _rsqrt_vreg, 0)
    pltpu.sync_copy(dinv_loc.at[pl.ds(0, NSL)], dinv_sh.at[sl])
    plsc.subcore_barrier()
    pltpu.sync_copy(dinv_sh, dinv_loc)

    def _main_chunk(ch, c):
        base = sid * EPT + ch * CH
        pltpu.sync_copy(rowp.at[pl.ds(base, CH)], row_b)
        pltpu.sync_copy(colp.at[pl.ds(base, CH)], col_b)
        pltpu.sync_copy(ewp.at[cid, pl.ds(base, CH)], ew_b)
        pltpu.async_copy(xp.at[row_b], xb, sem).wait()

        def _coeff_vreg(i, cc):
            s = pl.ds(i * 16, 16)
            dr = plsc.load_gather(dinv_loc, [row_b[s]])
            dc = plsc.load_gather(dinv_loc, [col_b[s]])
            coeff_b[s] = dr * jnp.abs(ew_b[s]) * dc
            return cc
        lax.fori_loop(0, CH // 16, _coeff_vreg, 0)

        def _scale_vreg(i, cc):
            cf16 = coeff_b[pl.ds(i * 16, 16)]
            for ln in range(16):
                e = i * 16 + ln
                cf = cf16[ln]
                for j in range(D // 16):
                    s = pl.ds(j * 16, 16)
                    xb[e, s] = xb[e, s] * cf
            return cc
        lax.fori_loop(0, CH // 16, _scale_vreg, 0)
        pltpu.sync_copy(xb, agg_sh.at[col_b], add=True)
        return c
    lax.fori_loop(0, NCH, _main_chunk, 0)
    plsc.subcore_barrier()

    for k in range(NSL // CH):
        r0 = sid * NSL + k * CH
        pltpu.sync_copy(agg_sh.at[pl.ds(r0, CH)], xb)
        pltpu.sync_copy(xb, out.at[cid, pl.ds(r0, CH)])


_sc_call = functools.partial(
    pl.kernel,
    out_type=jax.ShapeDtypeStruct((2, NPAD, D), jnp.float32),
    mesh=plsc.VectorSubcoreMesh(core_axis_name="c", subcore_axis_name="s"),
    compiler_params=pltpu.CompilerParams(needs_layout_passes=False),
    scratch_types=[
        pltpu.VMEM_SHARED((NPAD, D), jnp.float32),
        pltpu.VMEM_SHARED((NPAD,), jnp.float32),
        pltpu.VMEM_SHARED((NPAD,), jnp.float32),
        pltpu.VMEM((CH,), jnp.int32),
        pltpu.VMEM((CH,), jnp.int32),
        pltpu.VMEM((CH,), jnp.float32),
        pltpu.VMEM((CH,), jnp.float32),
        pltpu.VMEM((CH,), jnp.float32),
        pltpu.VMEM((NPAD,), jnp.float32),
        pltpu.VMEM((CH, D), jnp.float32),
        pltpu.SemaphoreType.DMA,
    ],
)(_sc_body)


def _mm_body(a_ref, w1_ref, w2_ref, b1_ref, b2_ref, o_ref):
    o_ref[:, :D] = (
        jnp.dot(a_ref[0], w1_ref[...], preferred_element_type=jnp.float32)
        + b1_ref[...]
    )
    o_ref[:, D:] = (
        jnp.dot(a_ref[1], w2_ref[...], preferred_element_type=jnp.float32)
        + b2_ref[...]
    )


_MB = 2000


def kernel(x, edge_index, edge_feat, W1, b1, W2, b2):
    idt = edge_index.dtype
    row = edge_index[0]
    col = edge_index[1]
    loop = jnp.arange(N, dtype=idt)
    padi = N + (jnp.arange(EPAD - E2, dtype=idt) % (NPAD - N))
    rowp = jnp.concatenate([row, loop, padi])
    colp = jnp.concatenate([col, loop, padi])
    onesN = jnp.ones((N,), jnp.float32)
    zpad = jnp.zeros((EPAD - E2,), jnp.float32)
    ewp = jnp.stack([
        jnp.concatenate([edge_feat[:, 0], onesN, zpad]),
        jnp.concatenate([edge_feat[:, 1], onesN, zpad]),
    ])
    xp = jnp.pad(x, ((0, NPAD - N), (0, 0)))

    agg = _sc_call(xp, rowp, colp, ewp)

    out = pl.pallas_call(
        _mm_body,
        grid=(N // _MB,),
        in_specs=[
            pl.BlockSpec((2, _MB, D), lambda i: (0, i, 0)),
            pl.BlockSpec((D, D), lambda i: (0, 0)),
            pl.BlockSpec((D, D), lambda i: (0, 0)),
            pl.BlockSpec((1, D), lambda i: (0, 0)),
            pl.BlockSpec((1, D), lambda i: (0, 0)),
        ],
        out_specs=pl.BlockSpec((_MB, 2 * D), lambda i: (i, 0)),
        out_shape=jax.ShapeDtypeStruct((N, 2 * D), jnp.float32),
    )(agg, W1, W2, b1.reshape(1, D), b2.reshape(1, D))
    return out


# speedup vs baseline: 30.0092x; 1.9482x over previous
import functools

import jax
import jax.numpy as jnp
from jax import lax
from jax.experimental import pallas as pl
from jax.experimental.pallas import tpu as pltpu
from jax.experimental.pallas import tpu_sc as plsc

N = 10000
E = 320000
D = 128

NTILES = 16
NSL = 640
NPAD = NTILES * NSL
CH = 128
NST = 8
GRP = NST * CH
E2 = E + N
NGRP = 21
EPT = NGRP * GRP
EPAD = NTILES * EPT


def _sc_body(xp, rowp, colp, ewp, out, agg_sh, nrm_sh,
             row3, col3, ew3, drow, dcol, dtmp, xb,
             gsem, ssem, rsem, csem):
    cid = lax.axis_index("c")
    sid = lax.axis_index("s")
    zero16 = jnp.zeros((16,), jnp.float32)
    RPT = EPT // CH

    def _zero_xb(i, c):
        for j in range(D // 16):
            xb[i, pl.ds(j * 16, 16)] = zero16
        return c
    lax.fori_loop(0, CH, _zero_xb, 0)
    for j in range(CH // 16):
        ew3[0, pl.ds(j * 16, 16)] = zero16
    for k in range(NSL // CH):
        pltpu.sync_copy(xb.at[pl.ds(0, CH)],
                        agg_sh.at[pl.ds(sid * NSL + k * CH, CH)])
        pltpu.sync_copy(ew3.at[0], nrm_sh.at[pl.ds(sid * NSL + k * CH, CH)])
    plsc.subcore_barrier()

    def _deg_group(g, c):
        rbase = sid * RPT + g * NST

        @pl.when(g > 0)
        def _drain():
            for j in range(NST):
                pltpu.make_async_copy(
                    ew3.at[j], nrm_sh.at[col3.at[j]], rsem.at[j]).wait()
        pltpu.sync_copy(colp.at[pl.ds(rbase, NST)], col3)
        pltpu.sync_copy(ewp.at[cid, pl.ds(rbase, NST)], ew3)
        for j in range(NST):
            def _absj(v, cc, j=j):
                s = pl.ds(v * 16, 16)
                ew3[j, s] = jnp.abs(ew3[j, s])
                return cc
            lax.fori_loop(0, CH // 16, _absj, 0)
        for j in range(NST):
            pltpu.async_copy(ew3.at[j], nrm_sh.at[col3.at[j]], rsem.at[j],
                             add=True)
        return c
    lax.fori_loop(0, NGRP, _deg_group, 0)
    for j in range(NST):
        pltpu.make_async_copy(ew3.at[j], nrm_sh.at[col3.at[j]],
                              rsem.at[j]).wait()
    plsc.subcore_barrier()

    sl = pl.ds(sid * NSL, NSL)
    pltpu.sync_copy(nrm_sh.at[sl], dtmp)

    def _rsqrt_vreg(i, c):
        s = pl.ds(i * 16, 16)
        d = dtmp[s]
        ii = lax.bitcast_convert_type(d, jnp.int32)
        ii = jnp.int32(0x5F3759DF) - (ii >> 1)
        y = lax.bitcast_convert_type(ii, jnp.float32)
        for _ in range(3):
            y = y * (1.5 - 0.5 * d * y * y)
        dtmp[s] = y
        return c
    lax.fori_loop(0, NSL // 16, _rsqrt_vreg, 0)
    pltpu.sync_copy(dtmp, nrm_sh.at[sl])
    plsc.subcore_barrier()

    def _main_group(g, c):
        rbase = sid * RPT + g * NST

        @pl.when(g > 0)
        def _drain():
            for p in range(2):
                pltpu.make_async_copy(
                    xb.at[pl.ds(p * CH, CH)], agg_sh.at[col3.at[0]],
                    ssem.at[p]).wait()
        pltpu.sync_copy(rowp.at[pl.ds(rbase, NST)], row3)
        pltpu.sync_copy(colp.at[pl.ds(rbase, NST)], col3)
        pltpu.sync_copy(ewp.at[cid, pl.ds(rbase, NST)], ew3)
        for j in range(NST):
            pltpu.async_copy(nrm_sh.at[row3.at[j]], drow.at[j], rsem.at[j])
            pltpu.async_copy(nrm_sh.at[col3.at[j]], dcol.at[j], csem.at[j])
        pltpu.async_copy(xp.at[row3.at[0]], xb.at[pl.ds(0, CH)], gsem.at[0])
        for k in range(NST):
            p = k % 2
            q = 1 - p
            if k + 1 < NST:
                if k >= 1:
                    pltpu.make_async_copy(
                        xb.at[pl.ds(q * CH, CH)], agg_sh.at[col3.at[0]],
                        ssem.at[q]).wait()
                pltpu.async_copy(xp.at[row3.at[k + 1]],
                                 xb.at[pl.ds(q * CH, CH)], gsem.at[q])
            pltpu.make_async_copy(xp.at[row3.at[k]],
                                  xb.at[pl.ds(p * CH, CH)], gsem.at[p]).wait()
            pltpu.make_async_copy(nrm_sh.at[row3.at[k]], drow.at[k],
                                  rsem.at[k]).wait()
            pltpu.make_async_copy(nrm_sh.at[col3.at[k]], dcol.at[k],
                                  csem.at[k]).wait()

            def _vreg(v, cc, k=k, p=p):
                s = pl.ds(v * 16, 16)
                cf16 = drow[k, s] * jnp.abs(ew3[k, s]) * dcol[k, s]
                for ln in range(16):
                    e = p * CH + v * 16 + ln
                    cf = cf16[ln]
                    for f in range(D // 16):
                        sf = pl.ds(f * 16, 16)
                        xb[e, sf] = xb[e, sf] * cf
                return cc
            lax.fori_loop(0, CH // 16, _vreg, 0)
            pltpu.async_copy(xb.at[pl.ds(p * CH, CH)], agg_sh.at[col3.at[k]],
                             ssem.at[p], add=True)
        return c
    lax.fori_loop(0, NGRP, _main_group, 0)
    for p in range(2):
        pltpu.make_async_copy(xb.at[pl.ds(p * CH, CH)],
                              agg_sh.at[col3.at[0]], ssem.at[p]).wait()
    plsc.subcore_barrier()

    for k in range(NSL // CH):
        r0 = sid * NSL + k * CH
        pltpu.sync_copy(agg_sh.at[pl.ds(r0, CH)], xb.at[pl.ds(0, CH)])
        pltpu.sync_copy(xb.at[pl.ds(0, CH)], out.at[cid, pl.ds(r0, CH)])


_sc_call = functools.partial(
    pl.kernel,
    out_type=jax.ShapeDtypeStruct((2, NPAD, D), jnp.float32),
    mesh=plsc.VectorSubcoreMesh(core_axis_name="c", subcore_axis_name="s"),
    compiler_params=pltpu.CompilerParams(needs_layout_passes=False),
    scratch_types=[
        pltpu.VMEM_SHARED((NPAD, D), jnp.float32),
        pltpu.VMEM_SHARED((NPAD,), jnp.float32),
        pltpu.VMEM((NST, CH), jnp.int32),
        pltpu.VMEM((NST, CH), jnp.int32),
        pltpu.VMEM((NST, CH), jnp.float32),
        pltpu.VMEM((NST, CH), jnp.float32),
        pltpu.VMEM((NST, CH), jnp.float32),
        pltpu.VMEM((NSL,), jnp.float32),
        pltpu.VMEM((2 * CH, D), jnp.float32),
        pltpu.SemaphoreType.DMA((2,)),
        pltpu.SemaphoreType.DMA((2,)),
        pltpu.SemaphoreType.DMA((NST,)),
        pltpu.SemaphoreType.DMA((NST,)),
    ],
)(_sc_body)


def _mm_body(a_ref, w1_ref, w2_ref, b1_ref, b2_ref, o_ref):
    o_ref[:, :D] = (
        jnp.dot(a_ref[0], w1_ref[...], preferred_element_type=jnp.float32)
        + b1_ref[...]
    )
    o_ref[:, D:] = (
        jnp.dot(a_ref[1], w2_ref[...], preferred_element_type=jnp.float32)
        + b2_ref[...]
    )


_MB = 2000


def kernel(x, edge_index, edge_feat, W1, b1, W2, b2):
    idt = edge_index.dtype
    row = edge_index[0]
    col = edge_index[1]
    loop = jnp.arange(N, dtype=idt)
    padi = N + (jnp.arange(EPAD - E2, dtype=idt) % (NPAD - N))
    rowp = jnp.concatenate([row, loop, padi]).reshape(EPAD // CH, CH)
    colp = jnp.concatenate([col, loop, padi]).reshape(EPAD // CH, CH)
    onesN = jnp.ones((N,), jnp.float32)
    zpad = jnp.zeros((EPAD - E2,), jnp.float32)
    ewp = jnp.stack([
        jnp.concatenate([edge_feat[:, 0], onesN, zpad]),
        jnp.concatenate([edge_feat[:, 1], onesN, zpad]),
    ]).reshape(2, EPAD // CH, CH)
    xp = jnp.pad(x, ((0, NPAD - N), (0, 0)))

    agg = _sc_call(xp, rowp, colp, ewp)

    out = pl.pallas_call(
        _mm_body,
        grid=(N // _MB,),
        in_specs=[
            pl.BlockSpec((2, _MB, D), lambda i: (0, i, 0)),
            pl.BlockSpec((D, D), lambda i: (0, 0)),
            pl.BlockSpec((D, D), lambda i: (0, 0)),
            pl.BlockSpec((1, D), lambda i: (0, 0)),
            pl.BlockSpec((1, D), lambda i: (0, 0)),
        ],
        out_specs=pl.BlockSpec((_MB, 2 * D), lambda i: (i, 0)),
        out_shape=jax.ShapeDtypeStruct((N, 2 * D), jnp.float32),
    )(agg, W1, W2, b1.reshape(1, D), b2.reshape(1, D))
    return out


# speedup vs baseline: 30.0299x; 1.0007x over previous
import functools

import jax
import jax.numpy as jnp
from jax import lax
from jax.experimental import pallas as pl
from jax.experimental.pallas import tpu as pltpu
from jax.experimental.pallas import tpu_sc as plsc

N = 10000
E = 320000
D = 128

NTILES = 16
NSL = 640
NPAD = NTILES * NSL
CH = 128
NST = 8
GRP = NST * CH
E2 = E + N
NGRP = 21
EPT = NGRP * GRP
EPAD = NTILES * EPT


def _sc_body(xp, rowp, colp, ewp, out, agg_sh, nrm_sh,
             row3, col3, ew3, drow, dcol, dtmp, xb,
             gsem, ssem, rsem, csem):
    cid = lax.axis_index("c")
    sid = lax.axis_index("s")
    zero16 = jnp.zeros((16,), jnp.float32)
    RPT = EPT // CH

    def _zero_xb(i, c):
        for j in range(D // 16):
            xb[i, pl.ds(j * 16, 16)] = zero16
        return c
    lax.fori_loop(0, CH, _zero_xb, 0)
    for j in range(CH // 16):
        ew3[0, pl.ds(j * 16, 16)] = zero16
    for k in range(NSL // CH):
        pltpu.sync_copy(xb.at[pl.ds(0, CH)],
                        agg_sh.at[pl.ds(sid * NSL + k * CH, CH)])
        pltpu.sync_copy(ew3.at[0], nrm_sh.at[pl.ds(sid * NSL + k * CH, CH)])
    plsc.subcore_barrier()

    scope = jax.named_scope

    def _deg_group(g, c):
        rbase = sid * RPT + g * NST

        @pl.when(g > 0)
        def _drain():
            for j in range(NST):
                pltpu.make_async_copy(
                    ew3.at[j], nrm_sh.at[col3.at[j]], rsem.at[j]).wait()
        pltpu.sync_copy(colp.at[pl.ds(rbase, NST)], col3)
        pltpu.sync_copy(ewp.at[cid, pl.ds(rbase, NST)], ew3)
        for j in range(NST):
            def _absj(v, cc, j=j):
                s = pl.ds(v * 16, 16)
                ew3[j, s] = jnp.abs(ew3[j, s])
                return cc
            lax.fori_loop(0, CH // 16, _absj, 0)
        for j in range(NST):
            pltpu.async_copy(ew3.at[j], nrm_sh.at[col3.at[j]], rsem.at[j],
                             add=True)
        return c
    with scope("p1_deg"):
        lax.fori_loop(0, NGRP, _deg_group, 0)
        for j in range(NST):
            pltpu.make_async_copy(ew3.at[j], nrm_sh.at[col3.at[j]],
                                  rsem.at[j]).wait()
        plsc.subcore_barrier()

    sl = pl.ds(sid * NSL, NSL)
    pltpu.sync_copy(nrm_sh.at[sl], dtmp)

    def _rsqrt_vreg(i, c):
        s = pl.ds(i * 16, 16)
        d = dtmp[s]
        ii = lax.bitcast_convert_type(d, jnp.int32)
        ii = jnp.int32(0x5F3759DF) - (ii >> 1)
        y = lax.bitcast_convert_type(ii, jnp.float32)
        for _ in range(3):
            y = y * (1.5 - 0.5 * d * y * y)
        dtmp[s] = y
        return c
    with scope("p2_rsqrt"):
        lax.fori_loop(0, NSL // 16, _rsqrt_vreg, 0)
        pltpu.sync_copy(dtmp, nrm_sh.at[sl])
        plsc.subcore_barrier()

    def _main_group(g, c):
        rbase = sid * RPT + g * NST

        @pl.when(g > 0)
        def _drain():
            for p in range(2):
                pltpu.make_async_copy(
                    xb.at[pl.ds(p * CH, CH)], agg_sh.at[col3.at[0]],
                    ssem.at[p]).wait()
        pltpu.sync_copy(rowp.at[pl.ds(rbase, NST)], row3)
        pltpu.sync_copy(colp.at[pl.ds(rbase, NST)], col3)
        pltpu.sync_copy(ewp.at[cid, pl.ds(rbase, NST)], ew3)
        for j in range(NST):
            pltpu.async_copy(nrm_sh.at[row3.at[j]], drow.at[j], rsem.at[j])
            pltpu.async_copy(nrm_sh.at[col3.at[j]], dcol.at[j], csem.at[j])
        pltpu.async_copy(xp.at[row3.at[0]], xb.at[pl.ds(0, CH)], gsem.at[0])
        for k in range(NST):
            p = k % 2
            q = 1 - p
            if k + 1 < NST:
                if k >= 1:
                    pltpu.make_async_copy(
                        xb.at[pl.ds(q * CH, CH)], agg_sh.at[col3.at[0]],
                        ssem.at[q]).wait()
                pltpu.async_copy(xp.at[row3.at[k + 1]],
                                 xb.at[pl.ds(q * CH, CH)], gsem.at[q])
            pltpu.make_async_copy(xp.at[row3.at[k]],
                                  xb.at[pl.ds(p * CH, CH)], gsem.at[p]).wait()
            pltpu.make_async_copy(nrm_sh.at[row3.at[k]], drow.at[k],
                                  rsem.at[k]).wait()
            pltpu.make_async_copy(nrm_sh.at[col3.at[k]], dcol.at[k],
                                  csem.at[k]).wait()

            def _vreg(v, cc, k=k, p=p):
                s = pl.ds(v * 16, 16)
                cf16 = drow[k, s] * jnp.abs(ew3[k, s]) * dcol[k, s]
                for ln in range(16):
                    e = p * CH + v * 16 + ln
                    cf = cf16[ln]
                    for f in range(D // 16):
                        sf = pl.ds(f * 16, 16)
                        xb[e, sf] = xb[e, sf] * cf
                return cc
            lax.fori_loop(0, CH // 16, _vreg, 0)
            pltpu.async_copy(xb.at[pl.ds(p * CH, CH)], agg_sh.at[col3.at[k]],
                             ssem.at[p], add=True)
        return c
    with scope("p3_main"):
        lax.fori_loop(0, NGRP, _main_group, 0)
        for p in range(2):
            pltpu.make_async_copy(xb.at[pl.ds(p * CH, CH)],
                                  agg_sh.at[col3.at[0]], ssem.at[p]).wait()
        plsc.subcore_barrier()

    with scope("p4_out"):
        for k in range(NSL // CH):
            r0 = sid * NSL + k * CH
            pltpu.sync_copy(agg_sh.at[pl.ds(r0, CH)], xb.at[pl.ds(0, CH)])
            pltpu.sync_copy(xb.at[pl.ds(0, CH)], out.at[cid, pl.ds(r0, CH)])


_sc_call = functools.partial(
    pl.kernel,
    out_type=jax.ShapeDtypeStruct((2, NPAD, D), jnp.float32),
    mesh=plsc.VectorSubcoreMesh(core_axis_name="c", subcore_axis_name="s"),
    compiler_params=pltpu.CompilerParams(needs_layout_passes=False),
    scratch_types=[
        pltpu.VMEM_SHARED((NPAD, D), jnp.float32),
        pltpu.VMEM_SHARED((NPAD,), jnp.float32),
        pltpu.VMEM((NST, CH), jnp.int32),
        pltpu.VMEM((NST, CH), jnp.int32),
        pltpu.VMEM((NST, CH), jnp.float32),
        pltpu.VMEM((NST, CH), jnp.float32),
        pltpu.VMEM((NST, CH), jnp.float32),
        pltpu.VMEM((NSL,), jnp.float32),
        pltpu.VMEM((2 * CH, D), jnp.float32),
        pltpu.SemaphoreType.DMA((2,)),
        pltpu.SemaphoreType.DMA((2,)),
        pltpu.SemaphoreType.DMA((NST,)),
        pltpu.SemaphoreType.DMA((NST,)),
    ],
)(_sc_body)


def _mm_body(a_ref, w1_ref, w2_ref, b1_ref, b2_ref, o_ref):
    o_ref[:, :D] = (
        jnp.dot(a_ref[0], w1_ref[...], preferred_element_type=jnp.float32)
        + b1_ref[...]
    )
    o_ref[:, D:] = (
        jnp.dot(a_ref[1], w2_ref[...], preferred_element_type=jnp.float32)
        + b2_ref[...]
    )


_MB = 2000


def kernel(x, edge_index, edge_feat, W1, b1, W2, b2):
    idt = edge_index.dtype
    row = edge_index[0]
    col = edge_index[1]
    loop = jnp.arange(N, dtype=idt)
    padi = N + (jnp.arange(EPAD - E2, dtype=idt) % (NPAD - N))
    rowp = jnp.concatenate([row, loop, padi]).reshape(EPAD // CH, CH)
    colp = jnp.concatenate([col, loop, padi]).reshape(EPAD // CH, CH)
    onesN = jnp.ones((N,), jnp.float32)
    zpad = jnp.zeros((EPAD - E2,), jnp.float32)
    ewp = jnp.stack([
        jnp.concatenate([edge_feat[:, 0], onesN, zpad]),
        jnp.concatenate([edge_feat[:, 1], onesN, zpad]),
    ]).reshape(2, EPAD // CH, CH)
    xp = jnp.pad(x, ((0, NPAD - N), (0, 0)))

    agg = _sc_call(xp, rowp, colp, ewp)

    out = pl.pallas_call(
        _mm_body,
        grid=(N // _MB,),
        in_specs=[
            pl.BlockSpec((2, _MB, D), lambda i: (0, i, 0)),
            pl.BlockSpec((D, D), lambda i: (0, 0)),
            pl.BlockSpec((D, D), lambda i: (0, 0)),
            pl.BlockSpec((1, D), lambda i: (0, 0)),
            pl.BlockSpec((1, D), lambda i: (0, 0)),
        ],
        out_specs=pl.BlockSpec((_MB, 2 * D), lambda i: (i, 0)),
        out_shape=jax.ShapeDtypeStruct((N, 2 * D), jnp.float32),
    )(agg, W1, W2, b1.reshape(1, D), b2.reshape(1, D))
    return out
